# baseline (speedup 1.0000x reference)
import functools

import jax
import jax.numpy as jnp
from jax import lax
from jax.experimental import pallas as pl
from jax.experimental.pallas import tpu as pltpu
from jax.experimental.pallas import tpu_sc as plsc

_NC = 2
_NS = 16
_NW = _NC * _NS


def _prep_body(tids_ref, lids_ref, te_ref, le_ref, wt_ref, wl_ref, pb_ref,
               rel_ref, w1r_ref, b1_ref, h0_ref, cp_ref):
  bn = tids_ref.shape[2]
  tids = tids_ref[0, 0, :]
  lids = lids_ref[0, 0, :]
  nt = te_ref.shape[0]
  nl = le_ref.shape[0]
  oh_t = (tids[:, None] == lax.broadcasted_iota(jnp.int32, (bn, nt), 1)
          ).astype(jnp.float32)
  oh_l = (lids[:, None] == lax.broadcasted_iota(jnp.int32, (bn, nl), 1)
          ).astype(jnp.float32)
  tt = jnp.dot(te_ref[...], wt_ref[...], preferred_element_type=jnp.float32)
  tl = jnp.dot(le_ref[...], wl_ref[...], preferred_element_type=jnp.float32)
  acc = (jnp.dot(oh_t, tt, preferred_element_type=jnp.float32)
         + jnp.dot(oh_l, tl, preferred_element_type=jnp.float32)
         + pb_ref[...])
  h0_ref[...] = jnp.maximum(acc, 0.0)

  @pl.when(pl.program_id(0) == 0)
  def _():
    for l in range(cp_ref.shape[0]):
      cp_ref[l] = (jnp.dot(rel_ref[...], w1r_ref[l],
                           preferred_element_type=jnp.float32)
                   + b1_ref[l][None, :])


def _tables_body(h_ref, wab_ref, t_ref):
  t_ref[0] = jnp.dot(h_ref[...], wab_ref[0],
                     preferred_element_type=jnp.float32)


def _msg_body(g_ref, rel_ref, cp_ref, w2_ref, b2_ref, out_ref):
  be = rel_ref.shape[2]
  nr = cp_ref.shape[0]
  rel = rel_ref[0, 0, :]
  oh = (rel[:, None] == lax.broadcasted_iota(jnp.int32, (be, nr), 1)
        ).astype(jnp.float32)
  hidden = jnp.maximum(
      g_ref[...] + jnp.dot(oh, cp_ref[...], preferred_element_type=jnp.float32),
      0.0)
  out_ref[...] = jnp.maximum(
      jnp.dot(hidden, w2_ref[...], preferred_element_type=jnp.float32)
      + b2_ref[...], 0.0)


def _upd_body(aggp_ref, h_ref, w_ref, b_ref, lg_ref, lb_ref, out_ref):
  agg = aggp_ref[0] + aggp_ref[1]
  u = h_ref[...] + jnp.maximum(
      jnp.dot(agg, w_ref[...], preferred_element_type=jnp.float32)
      + b_ref[...], 0.0)
  mu = jnp.mean(u, axis=-1, keepdims=True)
  var = jnp.mean((u - mu) ** 2, axis=-1, keepdims=True)
  out_ref[...] = (u - mu) * lax.rsqrt(var + 1e-5) * lg_ref[...] + lb_ref[...]


def _mean_body(h_ref, out_ref):
  out_ref[...] = jnp.mean(h_ref[...], axis=0, keepdims=True)


def _sc_gather_body(t_hbm, il_hbm, out_hbm,
                    idx0, idx1, gbuf0, gbuf1, obuf0, obuf1,
                    si0, si1, sg0, sg1, so0, so1,
                    *, epw, c, h):
  wid = lax.axis_index("s") * _NC + lax.axis_index("c")
  base = wid * epw
  nch = epw // c
  si = (si0, si1)
  sg = (sg0, sg1)
  so = (so0, so1)
  idxs = (idx0, idx1)
  gbuf = (gbuf0, gbuf1)
  obuf = (obuf0, obuf1)
  c2 = 2 * c

  def fire_idx(k, b):
    off = 2 * base + k * c2
    pltpu.async_copy(il_hbm.at[pl.ds(off, c2)], idxs[b], si[b])

  def wait_idx(k, b):
    off = 2 * base + k * c2
    pltpu.make_async_copy(il_hbm.at[pl.ds(off, c2)], idxs[b], si[b]).wait()

  def fire_gather(b):
    pltpu.async_copy(t_hbm.at[idxs[b]], gbuf[b], sg[b])

  def wait_gather(b):
    pltpu.make_async_copy(t_hbm.at[idxs[b]], gbuf[b], sg[b]).wait()

  def fire_out(k, b):
    off = base + k * c
    pltpu.async_copy(obuf[b], out_hbm.at[pl.ds(off, c)], so[b])

  def wait_out(k, b):
    off = base + k * c
    pltpu.make_async_copy(obuf[b], out_hbm.at[pl.ds(off, c)], so[b]).wait()

  def add(b):
    gb = gbuf[b]
    ob = obuf[b]

    def row(r, rc):
      for j in range(h // 16):
        sl = pl.ds(j * 16, 16)
        ob[r, sl] = gb[2 * r, sl] + gb[2 * r + 1, sl]
      return rc

    lax.fori_loop(0, c, row, 0, unroll=2)

  fire_idx(0, 0)
  wait_idx(0, 0)
  fire_gather(0)
  fire_idx(1, 1)

  def pair(p, carry):
    for b in range(2):
      k = 2 * p + b

      @pl.when(k < nch)
      def _():
        @pl.when(k + 1 < nch)
        def _():
          wait_idx(k + 1, 1 - b)
          fire_gather(1 - b)

        wait_gather(b)

        @pl.when(k + 2 < nch)
        def _():
          fire_idx(k + 2, b)

        @pl.when(k >= 2)
        def _():
          wait_out(k - 2, b)

        add(b)
        fire_out(k, b)
    return carry

  lax.fori_loop(0, (nch + 1) // 2, pair, 0)
  wait_out(nch - 2, (nch - 2) % 2)
  wait_out(nch - 1, (nch - 1) % 2)


def _sc_scatter_body(msg_hbm, dst_hbm, z_hbm, out_hbm,
                     idx0, idx1, buf0, buf1, agg_sh, si0, si1, sm0, sm1,
                     *, epw, c, n, cz):
  cc = lax.axis_index("c")
  ss = lax.axis_index("s")
  base = (cc * _NS + ss) * epw
  nch = epw // c
  nchn = n // cz
  npasses = (nchn + _NS - 1) // _NS

  for p in range(npasses):
    ck = ss + p * _NS

    @pl.when(ck < nchn)
    def _():
      pltpu.sync_copy(z_hbm.at[pl.ds(ck * cz, cz)],
                      agg_sh.at[pl.ds(ck * cz, cz)])
  plsc.subcore_barrier()

  si = (si0, si1)
  sm = (sm0, sm1)
  idxs = (idx0, idx1)
  bufs = (buf0, buf1)

  def stage(k, b):
    off = base + k * c
    pltpu.async_copy(dst_hbm.at[pl.ds(off, c)], idxs[b], si[b])
    pltpu.async_copy(msg_hbm.at[pl.ds(off, c)], bufs[b], sm[b])

  def wait_stage(k, b):
    off = base + k * c
    pltpu.make_async_copy(dst_hbm.at[pl.ds(off, c)], idxs[b], si[b]).wait()
    pltpu.make_async_copy(msg_hbm.at[pl.ds(off, c)], bufs[b], sm[b]).wait()

  stage(0, 0)

  def pair(p, carry):
    for b in range(2):
      k = 2 * p + b

      @pl.when(k < nch)
      def _():
        @pl.when(k + 1 < nch)
        def _():
          stage(k + 1, 1 - b)

        wait_stage(k, b)
        pltpu.sync_copy(bufs[b], agg_sh.at[idxs[b]], add=True)
    return carry

  lax.fori_loop(0, (nch + 1) // 2, pair, 0)
  plsc.subcore_barrier()

  for p in range(npasses):
    ck = ss + p * _NS

    @pl.when(ck < nchn)
    def _():
      pltpu.sync_copy(agg_sh.at[pl.ds(ck * cz, cz)],
                      out_hbm.at[cc, pl.ds(ck * cz, cz)])


def kernel(node_type_ids, label_ids, edge_index, edge_rel_ids, node_type_emb,
           label_emb, rel_emb, proj_W, proj_b, edge_W1, edge_b1, edge_W2,
           edge_b2, node_W, node_b, ln_g, ln_b):
  n, h = node_type_emb.shape[1], node_type_emb.shape[1]
  n = node_type_ids.shape[0]
  e = edge_rel_ids.shape[0]
  nlayers = edge_W1.shape[0]
  nt = node_type_emb.shape[0]
  nl = label_emb.shape[0]
  nr = rel_emb.shape[0]

  bn = 1000
  nbn = n // bn
  be = 2000
  nbe = e // be
  c = 80
  epw = e // _NW

  src = edge_index[0]
  dst = edge_index[1]
  idx_il = jnp.stack([src, dst + n], axis=1).ravel()
  rel3 = edge_rel_ids.reshape(nbe, 1, be)
  tids3 = node_type_ids.reshape(nbn, 1, bn)
  lids3 = label_ids.reshape(nbn, 1, bn)
  zeros_n = jnp.zeros((n, h), jnp.float32)

  wt = proj_W[:h]
  wl = proj_W[h:]
  pb = proj_b.reshape(1, h)
  w1r = edge_W1[:, 2 * h:, :]

  full = lambda shape: pl.BlockSpec(shape, lambda *a: tuple(0 for _ in shape))

  h0, cp = pl.pallas_call(
      _prep_body,
      grid=(nbn,),
      in_specs=[
          pl.BlockSpec((1, 1, bn), lambda i: (i, 0, 0)),
          pl.BlockSpec((1, 1, bn), lambda i: (i, 0, 0)),
          full((nt, h)),
          full((nl, h)),
          full((h, h)),
          full((h, h)),
          full((1, h)),
          full((nr, rel_emb.shape[1])),
          full((nlayers, rel_emb.shape[1], h)),
          full((nlayers, h)),
      ],
      out_specs=[
          pl.BlockSpec((bn, h), lambda i: (i, 0)),
          full((nlayers, nr, h)),
      ],
      out_shape=[
          jax.ShapeDtypeStruct((n, h), jnp.float32),
          jax.ShapeDtypeStruct((nlayers, nr, h), jnp.float32),
      ],
  )(tids3, lids3, node_type_emb, label_emb, wt, wl, pb, rel_emb, w1r, edge_b1)

  mesh = plsc.VectorSubcoreMesh(core_axis_name="c", subcore_axis_name="s",
                                num_cores=_NC, num_subcores=_NS)

  sc_gather = pl.kernel(
      functools.partial(_sc_gather_body, epw=epw, c=c, h=h),
      out_type=jax.ShapeDtypeStruct((e, h), jnp.float32),
      mesh=mesh,
      scratch_types=[
          pltpu.VMEM((2 * c,), jnp.int32),
          pltpu.VMEM((2 * c,), jnp.int32),
          pltpu.VMEM((2 * c, h), jnp.float32),
          pltpu.VMEM((2 * c, h), jnp.float32),
          pltpu.VMEM((c, h), jnp.float32),
          pltpu.VMEM((c, h), jnp.float32),
          pltpu.SemaphoreType.DMA,
          pltpu.SemaphoreType.DMA,
          pltpu.SemaphoreType.DMA,
          pltpu.SemaphoreType.DMA,
          pltpu.SemaphoreType.DMA,
          pltpu.SemaphoreType.DMA,
      ],
  )

  csc = 80
  sc_scatter = pl.kernel(
      functools.partial(_sc_scatter_body, epw=epw, c=csc, n=n, cz=400),
      out_type=jax.ShapeDtypeStruct((_NC, n, h), jnp.float32),
      mesh=mesh,
      scratch_types=[
          pltpu.VMEM((csc,), jnp.int32),
          pltpu.VMEM((csc,), jnp.int32),
          pltpu.VMEM((csc, h), jnp.float32),
          pltpu.VMEM((csc, h), jnp.float32),
          pltpu.VMEM_SHARED((n, h), jnp.float32),
          pltpu.SemaphoreType.DMA,
          pltpu.SemaphoreType.DMA,
          pltpu.SemaphoreType.DMA,
          pltpu.SemaphoreType.DMA,
      ],
  )

  tables_call = pl.pallas_call(
      _tables_body,
      grid=(2, nbn),
      in_specs=[
          pl.BlockSpec((bn, h), lambda p, i: (i, 0)),
          pl.BlockSpec((1, h, h), lambda p, i: (p, 0, 0)),
      ],
      out_specs=pl.BlockSpec((1, bn, h), lambda p, i: (p, i, 0)),
      out_shape=jax.ShapeDtypeStruct((2, n, h), jnp.float32),
  )

  msg_call = pl.pallas_call(
      _msg_body,
      grid=(nbe,),
      in_specs=[
          pl.BlockSpec((be, h), lambda i: (i, 0)),
          pl.BlockSpec((1, 1, be), lambda i: (i, 0, 0)),
          full((nr, h)),
          full((h, h)),
          full((1, h)),
      ],
      out_specs=pl.BlockSpec((be, h), lambda i: (i, 0)),
      out_shape=jax.ShapeDtypeStruct((e, h), jnp.float32),
  )

  upd_call = pl.pallas_call(
      _upd_body,
      grid=(nbn,),
      in_specs=[
          pl.BlockSpec((2, bn, h), lambda i: (0, i, 0)),
          pl.BlockSpec((bn, h), lambda i: (i, 0)),
          full((h, h)),
          full((1, h)),
          full((1, h)),
          full((1, h)),
      ],
      out_specs=pl.BlockSpec((bn, h), lambda i: (i, 0)),
      out_shape=jax.ShapeDtypeStruct((n, h), jnp.float32),
  )

  hcur = h0
  for l in range(nlayers):
    w1ab = edge_W1[l, :2 * h, :].reshape(2, h, h)
    tabs = tables_call(hcur, w1ab)
    t2 = tabs.reshape(2 * n, h)
    g = sc_gather(t2, idx_il)
    msg = msg_call(g, rel3, cp[l], edge_W2[l], edge_b2[l].reshape(1, h))
    aggp = sc_scatter(msg, dst, zeros_n)
    hcur = upd_call(aggp, hcur, node_W[l], node_b[l].reshape(1, h),
                    ln_g[l].reshape(1, h), ln_b[l].reshape(1, h))

  out = pl.pallas_call(
      _mean_body,
      in_specs=[pl.BlockSpec((n, h), lambda: (0, 0))],
      out_specs=pl.BlockSpec((1, h), lambda: (0, 0)),
      out_shape=jax.ShapeDtypeStruct((1, h), jnp.float32),
  )(hcur)
  return out.reshape(h)


# speedup vs baseline: 1.1993x; 1.1993x over previous
import functools

import jax
import numpy as np
import jax.numpy as jnp
from jax import lax
from jax.experimental import pallas as pl
from jax.experimental.pallas import tpu as pltpu
from jax.experimental.pallas import tpu_sc as plsc

_NC = 2
_NS = 16
_NW = _NC * _NS


def _prep_body(tids_ref, lids_ref, te_ref, le_ref, wt_ref, wl_ref, pb_ref,
               rel_ref, w1r_ref, b1_ref, h0_ref, cp_ref):
  bn = tids_ref.shape[2]
  tids = tids_ref[0, 0, :]
  lids = lids_ref[0, 0, :]
  nt = te_ref.shape[0]
  nl = le_ref.shape[0]
  oh_t = (tids[:, None] == lax.broadcasted_iota(jnp.int32, (bn, nt), 1)
          ).astype(jnp.float32)
  oh_l = (lids[:, None] == lax.broadcasted_iota(jnp.int32, (bn, nl), 1)
          ).astype(jnp.float32)
  tt = jnp.dot(te_ref[...], wt_ref[...], preferred_element_type=jnp.float32)
  tl = jnp.dot(le_ref[...], wl_ref[...], preferred_element_type=jnp.float32)
  acc = (jnp.dot(oh_t, tt, preferred_element_type=jnp.float32)
         + jnp.dot(oh_l, tl, preferred_element_type=jnp.float32)
         + pb_ref[...])
  h0_ref[...] = jnp.maximum(acc, 0.0)

  @pl.when(pl.program_id(0) == 0)
  def _():
    for l in range(cp_ref.shape[0]):
      cp_ref[l] = (jnp.dot(rel_ref[...], w1r_ref[l],
                           preferred_element_type=jnp.float32)
                   + b1_ref[l][None, :])


def _tables_body(h_ref, wab_ref, t_ref):
  t_ref[0] = jnp.dot(h_ref[...], wab_ref[0],
                     preferred_element_type=jnp.float32)


def _msg_body(g_ref, rel_ref, cp_ref, w2_ref, b2_ref, out_ref):
  be = rel_ref.shape[2]
  nr = cp_ref.shape[0]
  rel = rel_ref[0, 0, :]
  oh = (rel[:, None] == lax.broadcasted_iota(jnp.int32, (be, nr), 1)
        ).astype(jnp.float32)
  hidden = jnp.maximum(
      g_ref[...] + jnp.dot(oh, cp_ref[...], preferred_element_type=jnp.float32),
      0.0)
  out_ref[...] = jnp.maximum(
      jnp.dot(hidden, w2_ref[...], preferred_element_type=jnp.float32)
      + b2_ref[...], 0.0)


def _upd_body(aggp1_ref, aggp2_ref, h_ref, w_ref, b_ref, lg_ref, lb_ref,
              out_ref):
  agg = (aggp1_ref[0] + aggp1_ref[1]) + (aggp2_ref[0] + aggp2_ref[1])
  u = h_ref[...] + jnp.maximum(
      jnp.dot(agg, w_ref[...], preferred_element_type=jnp.float32)
      + b_ref[...], 0.0)
  mu = jnp.mean(u, axis=-1, keepdims=True)
  var = jnp.mean((u - mu) ** 2, axis=-1, keepdims=True)
  out_ref[...] = (u - mu) * lax.rsqrt(var + 1e-5) * lg_ref[...] + lb_ref[...]


def _mean_body(h_ref, out_ref):
  out_ref[...] = jnp.mean(h_ref[...], axis=0, keepdims=True)


def _sc_gather_body(t_hbm, src_hbm, dstn_hbm, out_hbm,
                    idxa_all, idxb_all, bufa0, bufa1, bufb0, bufb1,
                    sa0, sa1, sb0, sb1, so0, so1,
                    *, epw, c, h):
  wid = lax.axis_index("s") * _NC + lax.axis_index("c")
  base = wid * epw
  nch = epw // c
  sa = (sa0, sa1)
  sb = (sb0, sb1)
  so = (so0, so1)
  bufa = (bufa0, bufa1)
  bufb = (bufb0, bufb1)

  pltpu.sync_copy(src_hbm.at[pl.ds(base, epw)], idxa_all)
  pltpu.sync_copy(dstn_hbm.at[pl.ds(base, epw)], idxb_all)

  def stage(k, b):
    pltpu.async_copy(t_hbm.at[idxa_all.at[pl.ds(k * c, c)]], bufa[b], sa[b])
    pltpu.async_copy(t_hbm.at[idxb_all.at[pl.ds(k * c, c)]], bufb[b], sb[b])

  def wait_gather(k, b):
    pltpu.make_async_copy(
        t_hbm.at[idxa_all.at[pl.ds(k * c, c)]], bufa[b], sa[b]).wait()
    pltpu.make_async_copy(
        t_hbm.at[idxb_all.at[pl.ds(k * c, c)]], bufb[b], sb[b]).wait()

  def fire_out(k, b):
    off = base + k * c
    pltpu.async_copy(bufa[b], out_hbm.at[pl.ds(off, c)], so[b])

  def wait_out(k, b):
    off = base + k * c
    pltpu.make_async_copy(bufa[b], out_hbm.at[pl.ds(off, c)], so[b]).wait()

  def add(b):
    ba = bufa[b]
    bb = bufb[b]

    def row(r, rc):
      for j in range(h // 16):
        sl = pl.ds(j * 16, 16)
        ba[r, sl] = ba[r, sl] + bb[r, sl]
      return rc

    lax.fori_loop(0, c, row, 0, unroll=2)

  stage(0, 0)

  def pair(p, carry):
    for b in range(2):
      k = 2 * p + b

      @pl.when(k < nch)
      def _():
        @pl.when(k + 1 < nch)
        def _():
          @pl.when(k >= 1)
          def _():
            wait_out(k - 1, 1 - b)

          stage(k + 1, 1 - b)

        wait_gather(k, b)
        add(b)
        fire_out(k, b)
    return carry

  lax.fori_loop(0, (nch + 1) // 2, pair, 0)
  wait_out(nch - 2, (nch - 2) % 2)
  wait_out(nch - 1, (nch - 1) % 2)


def _sc_scatter_body(msg_hbm, dst_hbm, z_hbm, out_hbm,
                     idx0, idx1, buf0, buf1, agg_sh, si0, si1, sm0, sm1,
                     *, epw, c, n, cz):
  cc = lax.axis_index("c")
  ss = lax.axis_index("s")
  base = (cc * _NS + ss) * epw
  nch = epw // c
  nchn = n // cz
  npasses = (nchn + _NS - 1) // _NS

  for p in range(npasses):
    ck = ss + p * _NS

    @pl.when(ck < nchn)
    def _():
      pltpu.sync_copy(z_hbm.at[pl.ds(ck * cz, cz)],
                      agg_sh.at[pl.ds(ck * cz, cz)])
  plsc.subcore_barrier()

  si = (si0, si1)
  sm = (sm0, sm1)
  idxs = (idx0, idx1)
  bufs = (buf0, buf1)

  def stage(k, b):
    off = base + k * c
    pltpu.async_copy(dst_hbm.at[pl.ds(off, c)], idxs[b], si[b])
    pltpu.async_copy(msg_hbm.at[pl.ds(off, c)], bufs[b], sm[b])

  def wait_stage(k, b):
    off = base + k * c
    pltpu.make_async_copy(dst_hbm.at[pl.ds(off, c)], idxs[b], si[b]).wait()
    pltpu.make_async_copy(msg_hbm.at[pl.ds(off, c)], bufs[b], sm[b]).wait()

  stage(0, 0)

  def pair(p, carry):
    for b in range(2):
      k = 2 * p + b

      @pl.when(k < nch)
      def _():
        @pl.when(k + 1 < nch)
        def _():
          stage(k + 1, 1 - b)

        wait_stage(k, b)
        pltpu.sync_copy(bufs[b], agg_sh.at[idxs[b]], add=True)
    return carry

  lax.fori_loop(0, (nch + 1) // 2, pair, 0)
  plsc.subcore_barrier()

  for p in range(npasses):
    ck = ss + p * _NS

    @pl.when(ck < nchn)
    def _():
      pltpu.sync_copy(agg_sh.at[pl.ds(ck * cz, cz)],
                      out_hbm.at[cc, pl.ds(ck * cz, cz)])


def kernel(node_type_ids, label_ids, edge_index, edge_rel_ids, node_type_emb,
           label_emb, rel_emb, proj_W, proj_b, edge_W1, edge_b1, edge_W2,
           edge_b2, node_W, node_b, ln_g, ln_b):
  n, h = node_type_emb.shape[1], node_type_emb.shape[1]
  n = node_type_ids.shape[0]
  e = edge_rel_ids.shape[0]
  nlayers = edge_W1.shape[0]
  nt = node_type_emb.shape[0]
  nl = label_emb.shape[0]
  nr = rel_emb.shape[0]

  bn = 1000
  nbn = n // bn
  be = 1280
  c = 80
  e1 = (e // 2) // (_NW * c * 2) * (_NW * c * 2) * 2 // 2
  e1 = (e // 2) // (_NW * c) * (_NW * c)
  while e1 % be or (e - e1) % be or (e1 // _NW) % c or ((e - e1) // _NW) % c:
    e1 -= _NW * c
  e2 = e - e1

  src = edge_index[0]
  dst = edge_index[1]
  dstn = dst + n
  halves = []
  for lo, sz in ((0, e1), (e1, e2)):
    halves.append(dict(
        lo=lo, sz=sz,
        src=lax.slice(src, (lo,), (lo + sz,)),
        dstn=lax.slice(dstn, (lo,), (lo + sz,)),
        dst=lax.slice(dst, (lo,), (lo + sz,)),
        rel3=lax.slice(edge_rel_ids, (lo,), (lo + sz,)).reshape(
            sz // be, 1, be),
    ))
  tids3 = node_type_ids.reshape(nbn, 1, bn)
  lids3 = label_ids.reshape(nbn, 1, bn)
  zeros_n = jnp.zeros((n, h), jnp.float32)

  wt = proj_W[:h]
  wl = proj_W[h:]
  pb = proj_b.reshape(1, h)
  w1r = edge_W1[:, 2 * h:, :]

  full = lambda shape: pl.BlockSpec(shape, lambda *a: tuple(0 for _ in shape))

  h0, cp = pl.pallas_call(
      _prep_body,
      grid=(nbn,),
      in_specs=[
          pl.BlockSpec((1, 1, bn), lambda i: (i, 0, 0)),
          pl.BlockSpec((1, 1, bn), lambda i: (i, 0, 0)),
          full((nt, h)),
          full((nl, h)),
          full((h, h)),
          full((h, h)),
          full((1, h)),
          full((nr, rel_emb.shape[1])),
          full((nlayers, rel_emb.shape[1], h)),
          full((nlayers, h)),
      ],
      out_specs=[
          pl.BlockSpec((bn, h), lambda i: (i, 0)),
          full((nlayers, nr, h)),
      ],
      out_shape=[
          jax.ShapeDtypeStruct((n, h), jnp.float32),
          jax.ShapeDtypeStruct((nlayers, nr, h), jnp.float32),
      ],
  )(tids3, lids3, node_type_emb, label_emb, wt, wl, pb, rel_emb, w1r, edge_b1)

  mesh = plsc.VectorSubcoreMesh(core_axis_name="c", subcore_axis_name="s",
                                num_cores=_NC, num_subcores=_NS)

  def make_gather(sz):
    epw = sz // _NW
    return pl.kernel(
      functools.partial(_sc_gather_body, epw=epw, c=c, h=h),
      out_type=jax.ShapeDtypeStruct((sz, h), jnp.float32),
      mesh=mesh,
      scratch_types=[
          pltpu.VMEM((sz // _NW,), jnp.int32),
          pltpu.VMEM((sz // _NW,), jnp.int32),
          pltpu.VMEM((c, h), jnp.float32),
          pltpu.VMEM((c, h), jnp.float32),
          pltpu.VMEM((c, h), jnp.float32),
          pltpu.VMEM((c, h), jnp.float32),
          pltpu.SemaphoreType.DMA,
          pltpu.SemaphoreType.DMA,
          pltpu.SemaphoreType.DMA,
          pltpu.SemaphoreType.DMA,
          pltpu.SemaphoreType.DMA,
          pltpu.SemaphoreType.DMA,
      ],
  )

  csc = 80

  def make_scatter(sz):
    return pl.kernel(
        functools.partial(_sc_scatter_body, epw=sz // _NW, c=csc, n=n, cz=400),
        out_type=jax.ShapeDtypeStruct((_NC, n, h), jnp.float32),
        mesh=mesh,
        scratch_types=[
            pltpu.VMEM((csc,), jnp.int32),
            pltpu.VMEM((csc,), jnp.int32),
            pltpu.VMEM((csc, h), jnp.float32),
            pltpu.VMEM((csc, h), jnp.float32),
            pltpu.VMEM_SHARED((n, h), jnp.float32),
            pltpu.SemaphoreType.DMA,
            pltpu.SemaphoreType.DMA,
            pltpu.SemaphoreType.DMA,
            pltpu.SemaphoreType.DMA,
        ],
    )

  for hd in halves:
    hd["gather"] = make_gather(hd["sz"])
    hd["scatter"] = make_scatter(hd["sz"])

  tables_call = pl.pallas_call(
      _tables_body,
      grid=(2, nbn),
      in_specs=[
          pl.BlockSpec((bn, h), lambda p, i: (i, 0)),
          pl.BlockSpec((1, h, h), lambda p, i: (p, 0, 0)),
      ],
      out_specs=pl.BlockSpec((1, bn, h), lambda p, i: (p, i, 0)),
      out_shape=jax.ShapeDtypeStruct((2, n, h), jnp.float32),
  )

  def make_msg(sz):
    return pl.pallas_call(
        _msg_body,
        grid=(sz // be,),
        in_specs=[
            pl.BlockSpec((be, h), lambda i: (i, 0)),
            pl.BlockSpec((1, 1, be), lambda i: (i, 0, 0)),
            full((nr, h)),
            full((h, h)),
            full((1, h)),
        ],
        out_specs=pl.BlockSpec((be, h), lambda i: (i, 0)),
        out_shape=jax.ShapeDtypeStruct((sz, h), jnp.float32),
    )

  for hd in halves:
    hd["msg"] = make_msg(hd["sz"])

  upd_call = pl.pallas_call(
      _upd_body,
      grid=(nbn,),
      in_specs=[
          pl.BlockSpec((2, bn, h), lambda i: (0, i, 0)),
          pl.BlockSpec((2, bn, h), lambda i: (0, i, 0)),
          pl.BlockSpec((bn, h), lambda i: (i, 0)),
          full((h, h)),
          full((1, h)),
          full((1, h)),
          full((1, h)),
      ],
      out_specs=pl.BlockSpec((bn, h), lambda i: (i, 0)),
      out_shape=jax.ShapeDtypeStruct((n, h), jnp.float32),
  )

  hcur = h0
  for l in range(nlayers):
    w1ab = edge_W1[l, :2 * h, :].reshape(2, h, h)
    tabs = tables_call(hcur, w1ab)
    t2 = tabs.reshape(2 * n, h)
    aggps = []
    for hd in halves:
      g = hd["gather"](t2, hd["src"], hd["dstn"])
      msg = hd["msg"](g, hd["rel3"], cp[l], edge_W2[l],
                      edge_b2[l].reshape(1, h))
      aggps.append(hd["scatter"](msg, hd["dst"], zeros_n))
    hcur = upd_call(aggps[0], aggps[1], hcur, node_W[l],
                    node_b[l].reshape(1, h),
                    ln_g[l].reshape(1, h), ln_b[l].reshape(1, h))

  out = pl.pallas_call(
      _mean_body,
      in_specs=[pl.BlockSpec((n, h), lambda: (0, 0))],
      out_specs=pl.BlockSpec((1, h), lambda: (0, 0)),
      out_shape=jax.ShapeDtypeStruct((1, h), jnp.float32),
  )(hcur)
  return out.reshape(h)


# speedup vs baseline: 1.1995x; 1.0002x over previous
import functools

import jax
import numpy as np
import jax.numpy as jnp
from jax import lax
from jax.experimental import pallas as pl
from jax.experimental.pallas import tpu as pltpu
from jax.experimental.pallas import tpu_sc as plsc

_NC = 2
_NS = 16
_NW = _NC * _NS


def _prep_body(tids_ref, lids_ref, te_ref, le_ref, wt_ref, wl_ref, pb_ref,
               rel_ref, w1r_ref, b1_ref, h0_ref, cp_ref):
  bn = tids_ref.shape[2]
  tids = tids_ref[0, 0, :]
  lids = lids_ref[0, 0, :]
  nt = te_ref.shape[0]
  nl = le_ref.shape[0]
  oh_t = (tids[:, None] == lax.broadcasted_iota(jnp.int32, (bn, nt), 1)
          ).astype(jnp.float32)
  oh_l = (lids[:, None] == lax.broadcasted_iota(jnp.int32, (bn, nl), 1)
          ).astype(jnp.float32)
  tt = jnp.dot(te_ref[...], wt_ref[...], preferred_element_type=jnp.float32)
  tl = jnp.dot(le_ref[...], wl_ref[...], preferred_element_type=jnp.float32)
  acc = (jnp.dot(oh_t, tt, preferred_element_type=jnp.float32)
         + jnp.dot(oh_l, tl, preferred_element_type=jnp.float32)
         + pb_ref[...])
  h0_ref[...] = jnp.maximum(acc, 0.0)

  @pl.when(pl.program_id(0) == 0)
  def _():
    for l in range(cp_ref.shape[0]):
      cp_ref[l] = (jnp.dot(rel_ref[...], w1r_ref[l],
                           preferred_element_type=jnp.float32)
                   + b1_ref[l][None, :])


def _tables_body(h_ref, wab_ref, t_ref):
  t_ref[0] = jnp.dot(h_ref[...], wab_ref[0],
                     preferred_element_type=jnp.float32)


def _msg_body(g_ref, rel_ref, cp_ref, w2_ref, b2_ref, out_ref):
  be = rel_ref.shape[2]
  nr = cp_ref.shape[0]
  rel = rel_ref[0, 0, :]
  oh = (rel[:, None] == lax.broadcasted_iota(jnp.int32, (be, nr), 1)
        ).astype(jnp.float32)
  hidden = jnp.maximum(
      g_ref[...] + jnp.dot(oh, cp_ref[...], preferred_element_type=jnp.float32),
      0.0)
  out_ref[...] = jnp.maximum(
      jnp.dot(hidden, w2_ref[...], preferred_element_type=jnp.float32)
      + b2_ref[...], 0.0)


def _upd_body(aggp1_ref, aggp2_ref, h_ref, w_ref, b_ref, lg_ref, lb_ref,
              out_ref):
  agg = (aggp1_ref[0] + aggp1_ref[1]) + (aggp2_ref[0] + aggp2_ref[1])
  u = h_ref[...] + jnp.maximum(
      jnp.dot(agg, w_ref[...], preferred_element_type=jnp.float32)
      + b_ref[...], 0.0)
  mu = jnp.mean(u, axis=-1, keepdims=True)
  var = jnp.mean((u - mu) ** 2, axis=-1, keepdims=True)
  out_ref[...] = (u - mu) * lax.rsqrt(var + 1e-5) * lg_ref[...] + lb_ref[...]


def _mean_body(h_ref, out_ref):
  out_ref[...] = jnp.mean(h_ref[...], axis=0, keepdims=True)


def _sc_gather_body(t_hbm, src_hbm, dstn_hbm, out_hbm,
                    idxa_all, idxb_all, bufa0, bufa1, bufb0, bufb1,
                    sa0, sa1, sb0, sb1, so0, so1,
                    *, epw, c, h):
  wid = lax.axis_index("s") * _NC + lax.axis_index("c")
  base = wid * epw
  nch = epw // c
  sa = (sa0, sa1)
  sb = (sb0, sb1)
  so = (so0, so1)
  bufa = (bufa0, bufa1)
  bufb = (bufb0, bufb1)

  pltpu.sync_copy(src_hbm.at[pl.ds(base, epw)], idxa_all)
  pltpu.sync_copy(dstn_hbm.at[pl.ds(base, epw)], idxb_all)

  def stage(k, b):
    pltpu.async_copy(t_hbm.at[idxa_all.at[pl.ds(k * c, c)]], bufa[b], sa[b])
    pltpu.async_copy(t_hbm.at[idxb_all.at[pl.ds(k * c, c)]], bufb[b], sb[b])

  def wait_gather(k, b):
    pltpu.make_async_copy(
        t_hbm.at[idxa_all.at[pl.ds(k * c, c)]], bufa[b], sa[b]).wait()
    pltpu.make_async_copy(
        t_hbm.at[idxb_all.at[pl.ds(k * c, c)]], bufb[b], sb[b]).wait()

  def fire_out(k, b):
    off = base + k * c
    pltpu.async_copy(bufa[b], out_hbm.at[pl.ds(off, c)], so[b])

  def wait_out(k, b):
    off = base + k * c
    pltpu.make_async_copy(bufa[b], out_hbm.at[pl.ds(off, c)], so[b]).wait()

  def add(b):
    ba = bufa[b]
    bb = bufb[b]

    def row(r, rc):
      for j in range(h // 16):
        sl = pl.ds(j * 16, 16)
        ba[r, sl] = ba[r, sl] + bb[r, sl]
      return rc

    lax.fori_loop(0, c, row, 0, unroll=8)

  stage(0, 0)

  def pair(p, carry):
    for b in range(2):
      k = 2 * p + b

      @pl.when(k < nch)
      def _():
        @pl.when(k + 1 < nch)
        def _():
          @pl.when(k >= 1)
          def _():
            wait_out(k - 1, 1 - b)

          stage(k + 1, 1 - b)

        wait_gather(k, b)
        add(b)
        fire_out(k, b)
    return carry

  lax.fori_loop(0, (nch + 1) // 2, pair, 0)
  wait_out(nch - 2, (nch - 2) % 2)
  wait_out(nch - 1, (nch - 1) % 2)


def _sc_scatter_body(msg_hbm, dst_hbm, z_hbm, out_hbm,
                     idx0, idx1, buf0, buf1, agg_sh, si0, si1, sm0, sm1,
                     *, epw, c, n, cz):
  cc = lax.axis_index("c")
  ss = lax.axis_index("s")
  base = (cc * _NS + ss) * epw
  nch = epw // c
  nchn = n // cz
  npasses = (nchn + _NS - 1) // _NS

  for p in range(npasses):
    ck = ss + p * _NS

    @pl.when(ck < nchn)
    def _():
      pltpu.sync_copy(z_hbm.at[pl.ds(ck * cz, cz)],
                      agg_sh.at[pl.ds(ck * cz, cz)])
  plsc.subcore_barrier()

  si = (si0, si1)
  sm = (sm0, sm1)
  idxs = (idx0, idx1)
  bufs = (buf0, buf1)

  def stage(k, b):
    off = base + k * c
    pltpu.async_copy(dst_hbm.at[pl.ds(off, c)], idxs[b], si[b])
    pltpu.async_copy(msg_hbm.at[pl.ds(off, c)], bufs[b], sm[b])

  def wait_stage(k, b):
    off = base + k * c
    pltpu.make_async_copy(dst_hbm.at[pl.ds(off, c)], idxs[b], si[b]).wait()
    pltpu.make_async_copy(msg_hbm.at[pl.ds(off, c)], bufs[b], sm[b]).wait()

  stage(0, 0)

  def pair(p, carry):
    for b in range(2):
      k = 2 * p + b

      @pl.when(k < nch)
      def _():
        @pl.when(k + 1 < nch)
        def _():
          stage(k + 1, 1 - b)

        wait_stage(k, b)
        pltpu.sync_copy(bufs[b], agg_sh.at[idxs[b]], add=True)
    return carry

  lax.fori_loop(0, (nch + 1) // 2, pair, 0)
  plsc.subcore_barrier()

  for p in range(npasses):
    ck = ss + p * _NS

    @pl.when(ck < nchn)
    def _():
      pltpu.sync_copy(agg_sh.at[pl.ds(ck * cz, cz)],
                      out_hbm.at[cc, pl.ds(ck * cz, cz)])


def kernel(node_type_ids, label_ids, edge_index, edge_rel_ids, node_type_emb,
           label_emb, rel_emb, proj_W, proj_b, edge_W1, edge_b1, edge_W2,
           edge_b2, node_W, node_b, ln_g, ln_b):
  n, h = node_type_emb.shape[1], node_type_emb.shape[1]
  n = node_type_ids.shape[0]
  e = edge_rel_ids.shape[0]
  nlayers = edge_W1.shape[0]
  nt = node_type_emb.shape[0]
  nl = label_emb.shape[0]
  nr = rel_emb.shape[0]

  bn = 1000
  nbn = n // bn
  be = 1280
  c = 80
  e1 = (e // 2) // (_NW * c * 2) * (_NW * c * 2) * 2 // 2
  e1 = (e // 2) // (_NW * c) * (_NW * c)
  while e1 % be or (e - e1) % be or (e1 // _NW) % c or ((e - e1) // _NW) % c:
    e1 -= _NW * c
  e2 = e - e1

  src = edge_index[0]
  dst = edge_index[1]
  dstn = dst + n
  halves = []
  for lo, sz in ((0, e1), (e1, e2)):
    halves.append(dict(
        lo=lo, sz=sz,
        src=lax.slice(src, (lo,), (lo + sz,)),
        dstn=lax.slice(dstn, (lo,), (lo + sz,)),
        dst=lax.slice(dst, (lo,), (lo + sz,)),
        rel3=lax.slice(edge_rel_ids, (lo,), (lo + sz,)).reshape(
            sz // be, 1, be),
    ))
  tids3 = node_type_ids.reshape(nbn, 1, bn)
  lids3 = label_ids.reshape(nbn, 1, bn)
  zeros_n = jnp.zeros((n, h), jnp.float32)

  wt = proj_W[:h]
  wl = proj_W[h:]
  pb = proj_b.reshape(1, h)
  w1r = edge_W1[:, 2 * h:, :]

  full = lambda shape: pl.BlockSpec(shape, lambda *a: tuple(0 for _ in shape))

  h0, cp = pl.pallas_call(
      _prep_body,
      grid=(nbn,),
      in_specs=[
          pl.BlockSpec((1, 1, bn), lambda i: (i, 0, 0)),
          pl.BlockSpec((1, 1, bn), lambda i: (i, 0, 0)),
          full((nt, h)),
          full((nl, h)),
          full((h, h)),
          full((h, h)),
          full((1, h)),
          full((nr, rel_emb.shape[1])),
          full((nlayers, rel_emb.shape[1], h)),
          full((nlayers, h)),
      ],
      out_specs=[
          pl.BlockSpec((bn, h), lambda i: (i, 0)),
          full((nlayers, nr, h)),
      ],
      out_shape=[
          jax.ShapeDtypeStruct((n, h), jnp.float32),
          jax.ShapeDtypeStruct((nlayers, nr, h), jnp.float32),
      ],
  )(tids3, lids3, node_type_emb, label_emb, wt, wl, pb, rel_emb, w1r, edge_b1)

  mesh = plsc.VectorSubcoreMesh(core_axis_name="c", subcore_axis_name="s",
                                num_cores=_NC, num_subcores=_NS)

  def make_gather(sz):
    epw = sz // _NW
    return pl.kernel(
      functools.partial(_sc_gather_body, epw=epw, c=c, h=h),
      out_type=jax.ShapeDtypeStruct((sz, h), jnp.float32),
      mesh=mesh,
      scratch_types=[
          pltpu.VMEM((sz // _NW,), jnp.int32),
          pltpu.VMEM((sz // _NW,), jnp.int32),
          pltpu.VMEM((c, h), jnp.float32),
          pltpu.VMEM((c, h), jnp.float32),
          pltpu.VMEM((c, h), jnp.float32),
          pltpu.VMEM((c, h), jnp.float32),
          pltpu.SemaphoreType.DMA,
          pltpu.SemaphoreType.DMA,
          pltpu.SemaphoreType.DMA,
          pltpu.SemaphoreType.DMA,
          pltpu.SemaphoreType.DMA,
          pltpu.SemaphoreType.DMA,
      ],
  )

  csc = 80

  def make_scatter(sz):
    return pl.kernel(
        functools.partial(_sc_scatter_body, epw=sz // _NW, c=csc, n=n, cz=400),
        out_type=jax.ShapeDtypeStruct((_NC, n, h), jnp.float32),
        mesh=mesh,
        scratch_types=[
            pltpu.VMEM((csc,), jnp.int32),
            pltpu.VMEM((csc,), jnp.int32),
            pltpu.VMEM((csc, h), jnp.float32),
            pltpu.VMEM((csc, h), jnp.float32),
            pltpu.VMEM_SHARED((n, h), jnp.float32),
            pltpu.SemaphoreType.DMA,
            pltpu.SemaphoreType.DMA,
            pltpu.SemaphoreType.DMA,
            pltpu.SemaphoreType.DMA,
        ],
    )

  for hd in halves:
    hd["gather"] = make_gather(hd["sz"])
    hd["scatter"] = make_scatter(hd["sz"])

  tables_call = pl.pallas_call(
      _tables_body,
      grid=(2, nbn),
      in_specs=[
          pl.BlockSpec((bn, h), lambda p, i: (i, 0)),
          pl.BlockSpec((1, h, h), lambda p, i: (p, 0, 0)),
      ],
      out_specs=pl.BlockSpec((1, bn, h), lambda p, i: (p, i, 0)),
      out_shape=jax.ShapeDtypeStruct((2, n, h), jnp.float32),
  )

  def make_msg(sz):
    return pl.pallas_call(
        _msg_body,
        grid=(sz // be,),
        in_specs=[
            pl.BlockSpec((be, h), lambda i: (i, 0)),
            pl.BlockSpec((1, 1, be), lambda i: (i, 0, 0)),
            full((nr, h)),
            full((h, h)),
            full((1, h)),
        ],
        out_specs=pl.BlockSpec((be, h), lambda i: (i, 0)),
        out_shape=jax.ShapeDtypeStruct((sz, h), jnp.float32),
    )

  for hd in halves:
    hd["msg"] = make_msg(hd["sz"])

  upd_call = pl.pallas_call(
      _upd_body,
      grid=(nbn,),
      in_specs=[
          pl.BlockSpec((2, bn, h), lambda i: (0, i, 0)),
          pl.BlockSpec((2, bn, h), lambda i: (0, i, 0)),
          pl.BlockSpec((bn, h), lambda i: (i, 0)),
          full((h, h)),
          full((1, h)),
          full((1, h)),
          full((1, h)),
      ],
      out_specs=pl.BlockSpec((bn, h), lambda i: (i, 0)),
      out_shape=jax.ShapeDtypeStruct((n, h), jnp.float32),
  )

  hcur = h0
  for l in range(nlayers):
    w1ab = edge_W1[l, :2 * h, :].reshape(2, h, h)
    tabs = tables_call(hcur, w1ab)
    t2 = tabs.reshape(2 * n, h)
    aggps = []
    for hd in halves:
      g = hd["gather"](t2, hd["src"], hd["dstn"])
      msg = hd["msg"](g, hd["rel3"], cp[l], edge_W2[l],
                      edge_b2[l].reshape(1, h))
      aggps.append(hd["scatter"](msg, hd["dst"], zeros_n))
    hcur = upd_call(aggps[0], aggps[1], hcur, node_W[l],
                    node_b[l].reshape(1, h),
                    ln_g[l].reshape(1, h), ln_b[l].reshape(1, h))

  out = pl.pallas_call(
      _mean_body,
      in_specs=[pl.BlockSpec((n, h), lambda: (0, 0))],
      out_specs=pl.BlockSpec((1, h), lambda: (0, 0)),
      out_shape=jax.ShapeDtypeStruct((1, h), jnp.float32),
  )(hcur)
  return out.reshape(h)


# speedup vs baseline: 1.2170x; 1.0146x over previous
import functools

import jax
import numpy as np
import jax.numpy as jnp
from jax import lax
from jax.experimental import pallas as pl
from jax.experimental.pallas import tpu as pltpu
from jax.experimental.pallas import tpu_sc as plsc

_NC = 2
_NS = 16
_NW = _NC * _NS


def _prep_body(tids_ref, lids_ref, te_ref, le_ref, wt_ref, wl_ref, pb_ref,
               rel_ref, w1r_ref, b1_ref, h0_ref, cp_ref):
  bn = tids_ref.shape[2]
  tids = tids_ref[0, 0, :]
  lids = lids_ref[0, 0, :]
  nt = te_ref.shape[0]
  nl = le_ref.shape[0]
  oh_t = (tids[:, None] == lax.broadcasted_iota(jnp.int32, (bn, nt), 1)
          ).astype(jnp.float32)
  oh_l = (lids[:, None] == lax.broadcasted_iota(jnp.int32, (bn, nl), 1)
          ).astype(jnp.float32)
  tt = jnp.dot(te_ref[...], wt_ref[...], preferred_element_type=jnp.float32)
  tl = jnp.dot(le_ref[...], wl_ref[...], preferred_element_type=jnp.float32)
  acc = (jnp.dot(oh_t, tt, preferred_element_type=jnp.float32)
         + jnp.dot(oh_l, tl, preferred_element_type=jnp.float32)
         + pb_ref[...])
  h0_ref[...] = jnp.maximum(acc, 0.0)

  @pl.when(pl.program_id(0) == 0)
  def _():
    for l in range(cp_ref.shape[0]):
      cp_ref[l] = (jnp.dot(rel_ref[...], w1r_ref[l],
                           preferred_element_type=jnp.float32)
                   + b1_ref[l][None, :])


def _tables_body(h_ref, wab_ref, t_ref):
  t_ref[0] = jnp.dot(h_ref[...], wab_ref[0],
                     preferred_element_type=jnp.float32)


def _msg_body(g_ref, rel_ref, cp_ref, w2_ref, b2_ref, out_ref):
  be = rel_ref.shape[2]
  nr = cp_ref.shape[0]
  rel = rel_ref[0, 0, :]
  oh = (rel[:, None] == lax.broadcasted_iota(jnp.int32, (be, nr), 1)
        ).astype(jnp.float32)
  hidden = jnp.maximum(
      g_ref[...] + jnp.dot(oh, cp_ref[...], preferred_element_type=jnp.float32),
      0.0)
  out_ref[...] = jnp.maximum(
      jnp.dot(hidden, w2_ref[...], preferred_element_type=jnp.float32)
      + b2_ref[...], 0.0)


def _upd_body(aggp1_ref, aggp2_ref, h_ref, w_ref, b_ref, lg_ref, lb_ref,
              out_ref):
  agg = (aggp1_ref[0] + aggp1_ref[1]) + (aggp2_ref[0] + aggp2_ref[1])
  u = h_ref[...] + jnp.maximum(
      jnp.dot(agg, w_ref[...], preferred_element_type=jnp.float32)
      + b_ref[...], 0.0)
  mu = jnp.mean(u, axis=-1, keepdims=True)
  var = jnp.mean((u - mu) ** 2, axis=-1, keepdims=True)
  out_ref[...] = (u - mu) * lax.rsqrt(var + 1e-5) * lg_ref[...] + lb_ref[...]


def _mean_body(h_ref, out_ref):
  out_ref[...] = jnp.mean(h_ref[...], axis=0, keepdims=True)


def _sc_gather_body(t_hbm, src_hbm, dstn_hbm, out_hbm,
                    ia0, ia1, ib0, ib1, bufa0, bufa1, bufb0, bufb1,
                    obuf0, obuf1,
                    sia0, sia1, sib0, sib1, sa0, sa1, sb0, sb1, so0, so1,
                    *, epw, c, h):
  wid = lax.axis_index("s") * _NC + lax.axis_index("c")
  base = wid * epw
  nch = epw // c
  sia = (sia0, sia1)
  sib = (sib0, sib1)
  sa = (sa0, sa1)
  sb = (sb0, sb1)
  so = (so0, so1)
  idxa = (ia0, ia1)
  idxb = (ib0, ib1)
  bufa = (bufa0, bufa1)
  bufb = (bufb0, bufb1)
  obuf = (obuf0, obuf1)

  def fire_idx(k, b):
    off = base + k * c
    pltpu.async_copy(src_hbm.at[pl.ds(off, c)], idxa[b], sia[b])
    pltpu.async_copy(dstn_hbm.at[pl.ds(off, c)], idxb[b], sib[b])

  def wait_idx(k, b):
    off = base + k * c
    pltpu.make_async_copy(src_hbm.at[pl.ds(off, c)], idxa[b], sia[b]).wait()
    pltpu.make_async_copy(dstn_hbm.at[pl.ds(off, c)], idxb[b], sib[b]).wait()

  def fire_gather(b):
    pltpu.async_copy(t_hbm.at[idxa[b]], bufa[b], sa[b])
    pltpu.async_copy(t_hbm.at[idxb[b]], bufb[b], sb[b])

  def wait_gather(b):
    pltpu.make_async_copy(t_hbm.at[idxa[b]], bufa[b], sa[b]).wait()
    pltpu.make_async_copy(t_hbm.at[idxb[b]], bufb[b], sb[b]).wait()

  def fire_out(k, b):
    off = base + k * c
    pltpu.async_copy(obuf[b], out_hbm.at[pl.ds(off, c)], so[b])

  def wait_out(k, b):
    off = base + k * c
    pltpu.make_async_copy(obuf[b], out_hbm.at[pl.ds(off, c)], so[b]).wait()

  def add(b):
    ba = bufa[b]
    bb = bufb[b]
    ob = obuf[b]

    def row(r, rc):
      for j in range(h // 16):
        sl = pl.ds(j * 16, 16)
        ob[r, sl] = ba[r, sl] + bb[r, sl]
      return rc

    lax.fori_loop(0, c, row, 0, unroll=8)

  fire_idx(0, 0)
  wait_idx(0, 0)
  fire_gather(0)
  fire_idx(1, 1)

  def pair(p, carry):
    for b in range(2):
      k = 2 * p + b

      @pl.when(k < nch)
      def _():
        @pl.when(k + 1 < nch)
        def _():
          wait_idx(k + 1, 1 - b)
          fire_gather(1 - b)

        wait_gather(b)

        @pl.when(k + 2 < nch)
        def _():
          fire_idx(k + 2, b)

        @pl.when(k >= 2)
        def _():
          wait_out(k - 2, b)

        add(b)
        fire_out(k, b)
    return carry

  lax.fori_loop(0, (nch + 1) // 2, pair, 0)
  wait_out(nch - 2, (nch - 2) % 2)
  wait_out(nch - 1, (nch - 1) % 2)


def _sc_scatter_body(msg_hbm, dst_hbm, z_hbm, out_hbm,
                     idx0, idx1, buf0, buf1, agg_sh, si0, si1, sm0, sm1,
                     *, epw, c, n, cz):
  cc = lax.axis_index("c")
  ss = lax.axis_index("s")
  base = (cc * _NS + ss) * epw
  nch = epw // c
  nchn = n // cz
  npasses = (nchn + _NS - 1) // _NS

  for p in range(npasses):
    ck = ss + p * _NS

    @pl.when(ck < nchn)
    def _():
      pltpu.sync_copy(z_hbm.at[pl.ds(ck * cz, cz)],
                      agg_sh.at[pl.ds(ck * cz, cz)])
  plsc.subcore_barrier()

  si = (si0, si1)
  sm = (sm0, sm1)
  idxs = (idx0, idx1)
  bufs = (buf0, buf1)

  def stage(k, b):
    off = base + k * c
    pltpu.async_copy(dst_hbm.at[pl.ds(off, c)], idxs[b], si[b])
    pltpu.async_copy(msg_hbm.at[pl.ds(off, c)], bufs[b], sm[b])

  def wait_stage(k, b):
    off = base + k * c
    pltpu.make_async_copy(dst_hbm.at[pl.ds(off, c)], idxs[b], si[b]).wait()
    pltpu.make_async_copy(msg_hbm.at[pl.ds(off, c)], bufs[b], sm[b]).wait()

  stage(0, 0)

  def pair(p, carry):
    for b in range(2):
      k = 2 * p + b

      @pl.when(k < nch)
      def _():
        @pl.when(k + 1 < nch)
        def _():
          stage(k + 1, 1 - b)

        wait_stage(k, b)
        pltpu.sync_copy(bufs[b], agg_sh.at[idxs[b]], add=True)
    return carry

  lax.fori_loop(0, (nch + 1) // 2, pair, 0)
  plsc.subcore_barrier()

  for p in range(npasses):
    ck = ss + p * _NS

    @pl.when(ck < nchn)
    def _():
      pltpu.sync_copy(agg_sh.at[pl.ds(ck * cz, cz)],
                      out_hbm.at[cc, pl.ds(ck * cz, cz)])


def kernel(node_type_ids, label_ids, edge_index, edge_rel_ids, node_type_emb,
           label_emb, rel_emb, proj_W, proj_b, edge_W1, edge_b1, edge_W2,
           edge_b2, node_W, node_b, ln_g, ln_b):
  n, h = node_type_emb.shape[1], node_type_emb.shape[1]
  n = node_type_ids.shape[0]
  e = edge_rel_ids.shape[0]
  nlayers = edge_W1.shape[0]
  nt = node_type_emb.shape[0]
  nl = label_emb.shape[0]
  nr = rel_emb.shape[0]

  bn = 1000
  nbn = n // bn
  be = 1280
  c = 80
  e1 = (e // 2) // (_NW * c * 2) * (_NW * c * 2) * 2 // 2
  e1 = (e // 2) // (_NW * c) * (_NW * c)
  while e1 % be or (e - e1) % be or (e1 // _NW) % c or ((e - e1) // _NW) % c:
    e1 -= _NW * c
  e2 = e - e1

  src = edge_index[0]
  dst = edge_index[1]
  dstn = dst + n
  halves = []
  for lo, sz in ((0, e1), (e1, e2)):
    halves.append(dict(
        lo=lo, sz=sz,
        src=lax.slice(src, (lo,), (lo + sz,)),
        dstn=lax.slice(dstn, (lo,), (lo + sz,)),
        dst=lax.slice(dst, (lo,), (lo + sz,)),
        rel3=lax.slice(edge_rel_ids, (lo,), (lo + sz,)).reshape(
            sz // be, 1, be),
    ))
  tids3 = node_type_ids.reshape(nbn, 1, bn)
  lids3 = label_ids.reshape(nbn, 1, bn)
  zeros_n = jnp.zeros((n, h), jnp.float32)

  wt = proj_W[:h]
  wl = proj_W[h:]
  pb = proj_b.reshape(1, h)
  w1r = edge_W1[:, 2 * h:, :]

  full = lambda shape: pl.BlockSpec(shape, lambda *a: tuple(0 for _ in shape))

  h0, cp = pl.pallas_call(
      _prep_body,
      grid=(nbn,),
      in_specs=[
          pl.BlockSpec((1, 1, bn), lambda i: (i, 0, 0)),
          pl.BlockSpec((1, 1, bn), lambda i: (i, 0, 0)),
          full((nt, h)),
          full((nl, h)),
          full((h, h)),
          full((h, h)),
          full((1, h)),
          full((nr, rel_emb.shape[1])),
          full((nlayers, rel_emb.shape[1], h)),
          full((nlayers, h)),
      ],
      out_specs=[
          pl.BlockSpec((bn, h), lambda i: (i, 0)),
          full((nlayers, nr, h)),
      ],
      out_shape=[
          jax.ShapeDtypeStruct((n, h), jnp.float32),
          jax.ShapeDtypeStruct((nlayers, nr, h), jnp.float32),
      ],
  )(tids3, lids3, node_type_emb, label_emb, wt, wl, pb, rel_emb, w1r, edge_b1)

  mesh = plsc.VectorSubcoreMesh(core_axis_name="c", subcore_axis_name="s",
                                num_cores=_NC, num_subcores=_NS)

  def make_gather(sz):
    epw = sz // _NW
    return pl.kernel(
      functools.partial(_sc_gather_body, epw=epw, c=c, h=h),
      out_type=jax.ShapeDtypeStruct((sz, h), jnp.float32),
      mesh=mesh,
      scratch_types=[
          pltpu.VMEM((c,), jnp.int32),
          pltpu.VMEM((c,), jnp.int32),
          pltpu.VMEM((c,), jnp.int32),
          pltpu.VMEM((c,), jnp.int32),
          pltpu.VMEM((c, h), jnp.float32),
          pltpu.VMEM((c, h), jnp.float32),
          pltpu.VMEM((c, h), jnp.float32),
          pltpu.VMEM((c, h), jnp.float32),
          pltpu.VMEM((c, h), jnp.float32),
          pltpu.VMEM((c, h), jnp.float32),
          pltpu.SemaphoreType.DMA,
          pltpu.SemaphoreType.DMA,
          pltpu.SemaphoreType.DMA,
          pltpu.SemaphoreType.DMA,
          pltpu.SemaphoreType.DMA,
          pltpu.SemaphoreType.DMA,
          pltpu.SemaphoreType.DMA,
          pltpu.SemaphoreType.DMA,
          pltpu.SemaphoreType.DMA,
          pltpu.SemaphoreType.DMA,
      ],
  )

  csc = 80

  def make_scatter(sz):
    return pl.kernel(
        functools.partial(_sc_scatter_body, epw=sz // _NW, c=csc, n=n, cz=400),
        out_type=jax.ShapeDtypeStruct((_NC, n, h), jnp.float32),
        mesh=mesh,
        scratch_types=[
            pltpu.VMEM((csc,), jnp.int32),
            pltpu.VMEM((csc,), jnp.int32),
            pltpu.VMEM((csc, h), jnp.float32),
            pltpu.VMEM((csc, h), jnp.float32),
            pltpu.VMEM_SHARED((n, h), jnp.float32),
            pltpu.SemaphoreType.DMA,
            pltpu.SemaphoreType.DMA,
            pltpu.SemaphoreType.DMA,
            pltpu.SemaphoreType.DMA,
        ],
    )

  for hd in halves:
    hd["gather"] = make_gather(hd["sz"])
    hd["scatter"] = make_scatter(hd["sz"])

  tables_call = pl.pallas_call(
      _tables_body,
      grid=(2, nbn),
      in_specs=[
          pl.BlockSpec((bn, h), lambda p, i: (i, 0)),
          pl.BlockSpec((1, h, h), lambda p, i: (p, 0, 0)),
      ],
      out_specs=pl.BlockSpec((1, bn, h), lambda p, i: (p, i, 0)),
      out_shape=jax.ShapeDtypeStruct((2, n, h), jnp.float32),
  )

  def make_msg(sz):
    return pl.pallas_call(
        _msg_body,
        grid=(sz // be,),
        in_specs=[
            pl.BlockSpec((be, h), lambda i: (i, 0)),
            pl.BlockSpec((1, 1, be), lambda i: (i, 0, 0)),
            full((nr, h)),
            full((h, h)),
            full((1, h)),
        ],
        out_specs=pl.BlockSpec((be, h), lambda i: (i, 0)),
        out_shape=jax.ShapeDtypeStruct((sz, h), jnp.float32),
    )

  for hd in halves:
    hd["msg"] = make_msg(hd["sz"])

  upd_call = pl.pallas_call(
      _upd_body,
      grid=(nbn,),
      in_specs=[
          pl.BlockSpec((2, bn, h), lambda i: (0, i, 0)),
          pl.BlockSpec((2, bn, h), lambda i: (0, i, 0)),
          pl.BlockSpec((bn, h), lambda i: (i, 0)),
          full((h, h)),
          full((1, h)),
          full((1, h)),
          full((1, h)),
      ],
      out_specs=pl.BlockSpec((bn, h), lambda i: (i, 0)),
      out_shape=jax.ShapeDtypeStruct((n, h), jnp.float32),
  )

  hcur = h0
  for l in range(nlayers):
    w1ab = edge_W1[l, :2 * h, :].reshape(2, h, h)
    tabs = tables_call(hcur, w1ab)
    t2 = tabs.reshape(2 * n, h)
    aggps = []
    for hd in halves:
      g = hd["gather"](t2, hd["src"], hd["dstn"])
      msg = hd["msg"](g, hd["rel3"], cp[l], edge_W2[l],
                      edge_b2[l].reshape(1, h))
      aggps.append(hd["scatter"](msg, hd["dst"], zeros_n))
    hcur = upd_call(aggps[0], aggps[1], hcur, node_W[l],
                    node_b[l].reshape(1, h),
                    ln_g[l].reshape(1, h), ln_b[l].reshape(1, h))

  out = pl.pallas_call(
      _mean_body,
      in_specs=[pl.BlockSpec((n, h), lambda: (0, 0))],
      out_specs=pl.BlockSpec((1, h), lambda: (0, 0)),
      out_shape=jax.ShapeDtypeStruct((1, h), jnp.float32),
  )(hcur)
  return out.reshape(h)


# speedup vs baseline: 1.2455x; 1.0234x over previous
import functools

import jax
import numpy as np
import jax.numpy as jnp
from jax import lax
from jax.experimental import pallas as pl
from jax.experimental.pallas import tpu as pltpu
from jax.experimental.pallas import tpu_sc as plsc

_NC = 2
_NS = 16
_NW = _NC * _NS


def _prep_body(tids_ref, lids_ref, te_ref, le_ref, wt_ref, wl_ref, pb_ref,
               rel_ref, w1r_ref, b1_ref, w1ab_ref, h0_ref, cp_ref, t_ref):
  bn = tids_ref.shape[2]
  tids = tids_ref[0, 0, :]
  lids = lids_ref[0, 0, :]
  nt = te_ref.shape[0]
  nl = le_ref.shape[0]
  oh_t = (tids[:, None] == lax.broadcasted_iota(jnp.int32, (bn, nt), 1)
          ).astype(jnp.float32)
  oh_l = (lids[:, None] == lax.broadcasted_iota(jnp.int32, (bn, nl), 1)
          ).astype(jnp.float32)
  tt = jnp.dot(te_ref[...], wt_ref[...], preferred_element_type=jnp.float32)
  tl = jnp.dot(le_ref[...], wl_ref[...], preferred_element_type=jnp.float32)
  acc = (jnp.dot(oh_t, tt, preferred_element_type=jnp.float32)
         + jnp.dot(oh_l, tl, preferred_element_type=jnp.float32)
         + pb_ref[...])
  h0 = jnp.maximum(acc, 0.0)
  h0_ref[...] = h0
  t_ref[0] = jnp.dot(h0, w1ab_ref[0], preferred_element_type=jnp.float32)
  t_ref[1] = jnp.dot(h0, w1ab_ref[1], preferred_element_type=jnp.float32)

  @pl.when(pl.program_id(0) == 0)
  def _():
    for l in range(cp_ref.shape[0]):
      cp_ref[l] = (jnp.dot(rel_ref[...], w1r_ref[l],
                           preferred_element_type=jnp.float32)
                   + b1_ref[l][None, :])


def _tables_body(h_ref, wab_ref, t_ref):
  t_ref[0] = jnp.dot(h_ref[...], wab_ref[0],
                     preferred_element_type=jnp.float32)


def _msg_body(g_ref, rel_ref, cp_ref, w2_ref, b2_ref, out_ref):
  be = rel_ref.shape[2]
  nr = cp_ref.shape[0]
  rel = rel_ref[0, 0, :]
  oh = (rel[:, None] == lax.broadcasted_iota(jnp.int32, (be, nr), 1)
        ).astype(jnp.float32)
  hidden = jnp.maximum(
      g_ref[...] + jnp.dot(oh, cp_ref[...], preferred_element_type=jnp.float32),
      0.0)
  out_ref[...] = jnp.maximum(
      jnp.dot(hidden, w2_ref[...], preferred_element_type=jnp.float32)
      + b2_ref[...], 0.0)


def _upd_body(aggp1_ref, aggp2_ref, h_ref, w_ref, b_ref, lg_ref, lb_ref,
              w1ab_ref, out_ref, t_ref):
  agg = (aggp1_ref[0] + aggp1_ref[1]) + (aggp2_ref[0] + aggp2_ref[1])
  u = h_ref[...] + jnp.maximum(
      jnp.dot(agg, w_ref[...], preferred_element_type=jnp.float32)
      + b_ref[...], 0.0)
  mu = jnp.mean(u, axis=-1, keepdims=True)
  var = jnp.mean((u - mu) ** 2, axis=-1, keepdims=True)
  hn = (u - mu) * lax.rsqrt(var + 1e-5) * lg_ref[...] + lb_ref[...]
  out_ref[...] = hn
  t_ref[0] = jnp.dot(hn, w1ab_ref[0], preferred_element_type=jnp.float32)
  t_ref[1] = jnp.dot(hn, w1ab_ref[1], preferred_element_type=jnp.float32)


def _updlast_body(aggp1_ref, aggp2_ref, h_ref, w_ref, b_ref, lg_ref, lb_ref,
                  out_ref, m_ref):
  agg = (aggp1_ref[0] + aggp1_ref[1]) + (aggp2_ref[0] + aggp2_ref[1])
  u = h_ref[...] + jnp.maximum(
      jnp.dot(agg, w_ref[...], preferred_element_type=jnp.float32)
      + b_ref[...], 0.0)
  mu = jnp.mean(u, axis=-1, keepdims=True)
  var = jnp.mean((u - mu) ** 2, axis=-1, keepdims=True)
  hn = (u - mu) * lax.rsqrt(var + 1e-5) * lg_ref[...] + lb_ref[...]
  out_ref[...] = hn

  @pl.when(pl.program_id(0) == 0)
  def _():
    m_ref[...] = jnp.zeros_like(m_ref)

  m_ref[...] += jnp.sum(hn, axis=0, keepdims=True)


def _mean_body(h_ref, out_ref):
  out_ref[...] = jnp.mean(h_ref[...], axis=0, keepdims=True)


def _sc_gather_body(t_hbm, src_hbm, dstn_hbm, out_hbm,
                    ia0, ia1, ib0, ib1, bufa0, bufa1, bufb0, bufb1,
                    obuf0, obuf1,
                    sia0, sia1, sib0, sib1, sa0, sa1, sb0, sb1, so0, so1,
                    *, epw, c, h):
  wid = lax.axis_index("s") * _NC + lax.axis_index("c")
  base = wid * epw
  nch = epw // c
  sia = (sia0, sia1)
  sib = (sib0, sib1)
  sa = (sa0, sa1)
  sb = (sb0, sb1)
  so = (so0, so1)
  idxa = (ia0, ia1)
  idxb = (ib0, ib1)
  bufa = (bufa0, bufa1)
  bufb = (bufb0, bufb1)
  obuf = (obuf0, obuf1)

  def fire_idx(k, b):
    off = base + k * c
    pltpu.async_copy(src_hbm.at[pl.ds(off, c)], idxa[b], sia[b])
    pltpu.async_copy(dstn_hbm.at[pl.ds(off, c)], idxb[b], sib[b])

  def wait_idx(k, b):
    off = base + k * c
    pltpu.make_async_copy(src_hbm.at[pl.ds(off, c)], idxa[b], sia[b]).wait()
    pltpu.make_async_copy(dstn_hbm.at[pl.ds(off, c)], idxb[b], sib[b]).wait()

  def fire_gather(b):
    pltpu.async_copy(t_hbm.at[idxa[b]], bufa[b], sa[b])
    pltpu.async_copy(t_hbm.at[idxb[b]], bufb[b], sb[b])

  def wait_gather(b):
    pltpu.make_async_copy(t_hbm.at[idxa[b]], bufa[b], sa[b]).wait()
    pltpu.make_async_copy(t_hbm.at[idxb[b]], bufb[b], sb[b]).wait()

  def fire_out(k, b):
    off = base + k * c
    pltpu.async_copy(obuf[b], out_hbm.at[pl.ds(off, c)], so[b])

  def wait_out(k, b):
    off = base + k * c
    pltpu.make_async_copy(obuf[b], out_hbm.at[pl.ds(off, c)], so[b]).wait()

  def add(b):
    ba = bufa[b]
    bb = bufb[b]
    ob = obuf[b]

    def row(r, rc):
      for j in range(h // 16):
        sl = pl.ds(j * 16, 16)
        ob[r, sl] = ba[r, sl] + bb[r, sl]
      return rc

    lax.fori_loop(0, c, row, 0, unroll=8)

  fire_idx(0, 0)
  wait_idx(0, 0)
  fire_gather(0)
  fire_idx(1, 1)

  def pair(p, carry):
    for b in range(2):
      k = 2 * p + b

      @pl.when(k < nch)
      def _():
        @pl.when(k + 1 < nch)
        def _():
          wait_idx(k + 1, 1 - b)
          fire_gather(1 - b)

        wait_gather(b)

        @pl.when(k + 2 < nch)
        def _():
          fire_idx(k + 2, b)

        @pl.when(k >= 2)
        def _():
          wait_out(k - 2, b)

        add(b)
        fire_out(k, b)
    return carry

  lax.fori_loop(0, (nch + 1) // 2, pair, 0)
  wait_out(nch - 2, (nch - 2) % 2)
  wait_out(nch - 1, (nch - 1) % 2)


def _sc_scatter_body(msg_hbm, dst_hbm, z_hbm, out_hbm,
                     idx0, idx1, buf0, buf1, agg_sh, si0, si1, sm0, sm1,
                     *, epw, c, n, cz):
  cc = lax.axis_index("c")
  ss = lax.axis_index("s")
  base = (cc * _NS + ss) * epw
  nch = epw // c
  nchn = n // cz
  npasses = (nchn + _NS - 1) // _NS

  for p in range(npasses):
    ck = ss + p * _NS

    @pl.when(ck < nchn)
    def _():
      pltpu.sync_copy(z_hbm.at[pl.ds(ck * cz, cz)],
                      agg_sh.at[pl.ds(ck * cz, cz)])
  plsc.subcore_barrier()

  si = (si0, si1)
  sm = (sm0, sm1)
  idxs = (idx0, idx1)
  bufs = (buf0, buf1)

  def stage(k, b):
    off = base + k * c
    pltpu.async_copy(dst_hbm.at[pl.ds(off, c)], idxs[b], si[b])
    pltpu.async_copy(msg_hbm.at[pl.ds(off, c)], bufs[b], sm[b])

  def wait_stage(k, b):
    off = base + k * c
    pltpu.make_async_copy(dst_hbm.at[pl.ds(off, c)], idxs[b], si[b]).wait()
    pltpu.make_async_copy(msg_hbm.at[pl.ds(off, c)], bufs[b], sm[b]).wait()

  stage(0, 0)

  def pair(p, carry):
    for b in range(2):
      k = 2 * p + b

      @pl.when(k < nch)
      def _():
        @pl.when(k + 1 < nch)
        def _():
          stage(k + 1, 1 - b)

        wait_stage(k, b)
        pltpu.sync_copy(bufs[b], agg_sh.at[idxs[b]], add=True)
    return carry

  lax.fori_loop(0, (nch + 1) // 2, pair, 0)
  plsc.subcore_barrier()

  for p in range(npasses):
    ck = ss + p * _NS

    @pl.when(ck < nchn)
    def _():
      pltpu.sync_copy(agg_sh.at[pl.ds(ck * cz, cz)],
                      out_hbm.at[cc, pl.ds(ck * cz, cz)])


def kernel(node_type_ids, label_ids, edge_index, edge_rel_ids, node_type_emb,
           label_emb, rel_emb, proj_W, proj_b, edge_W1, edge_b1, edge_W2,
           edge_b2, node_W, node_b, ln_g, ln_b):
  n, h = node_type_emb.shape[1], node_type_emb.shape[1]
  n = node_type_ids.shape[0]
  e = edge_rel_ids.shape[0]
  nlayers = edge_W1.shape[0]
  nt = node_type_emb.shape[0]
  nl = label_emb.shape[0]
  nr = rel_emb.shape[0]

  bn = 1000
  nbn = n // bn
  be = 1280
  c = 80
  e1 = (e // 2) // (_NW * c * 2) * (_NW * c * 2) * 2 // 2
  e1 = (e // 2) // (_NW * c) * (_NW * c)
  while e1 % be or (e - e1) % be or (e1 // _NW) % c or ((e - e1) // _NW) % c:
    e1 -= _NW * c
  e2 = e - e1

  src = edge_index[0]
  dst = edge_index[1]
  dstn = dst + n
  halves = []
  for lo, sz in ((0, e1), (e1, e2)):
    halves.append(dict(
        lo=lo, sz=sz,
        src=lax.slice(src, (lo,), (lo + sz,)),
        dstn=lax.slice(dstn, (lo,), (lo + sz,)),
        dst=lax.slice(dst, (lo,), (lo + sz,)),
        rel3=lax.slice(edge_rel_ids, (lo,), (lo + sz,)).reshape(
            sz // be, 1, be),
    ))
  tids3 = node_type_ids.reshape(nbn, 1, bn)
  lids3 = label_ids.reshape(nbn, 1, bn)
  zeros_n = jnp.zeros((n, h), jnp.float32)

  wt = proj_W[:h]
  wl = proj_W[h:]
  pb = proj_b.reshape(1, h)
  w1r = edge_W1[:, 2 * h:, :]

  full = lambda shape: pl.BlockSpec(shape, lambda *a: tuple(0 for _ in shape))

  h0, cp, tabs = pl.pallas_call(
      _prep_body,
      grid=(nbn,),
      in_specs=[
          pl.BlockSpec((1, 1, bn), lambda i: (i, 0, 0)),
          pl.BlockSpec((1, 1, bn), lambda i: (i, 0, 0)),
          full((nt, h)),
          full((nl, h)),
          full((h, h)),
          full((h, h)),
          full((1, h)),
          full((nr, rel_emb.shape[1])),
          full((nlayers, rel_emb.shape[1], h)),
          full((nlayers, h)),
          full((2, h, h)),
      ],
      out_specs=[
          pl.BlockSpec((bn, h), lambda i: (i, 0)),
          full((nlayers, nr, h)),
          pl.BlockSpec((2, bn, h), lambda i: (0, i, 0)),
      ],
      out_shape=[
          jax.ShapeDtypeStruct((n, h), jnp.float32),
          jax.ShapeDtypeStruct((nlayers, nr, h), jnp.float32),
          jax.ShapeDtypeStruct((2, n, h), jnp.float32),
      ],
  )(tids3, lids3, node_type_emb, label_emb, wt, wl, pb, rel_emb, w1r, edge_b1,
    edge_W1[0, :2 * h, :].reshape(2, h, h))

  mesh = plsc.VectorSubcoreMesh(core_axis_name="c", subcore_axis_name="s",
                                num_cores=_NC, num_subcores=_NS)

  def make_gather(sz):
    epw = sz // _NW
    return pl.kernel(
      functools.partial(_sc_gather_body, epw=epw, c=c, h=h),
      out_type=jax.ShapeDtypeStruct((sz, h), jnp.float32),
      mesh=mesh,
      scratch_types=[
          pltpu.VMEM((c,), jnp.int32),
          pltpu.VMEM((c,), jnp.int32),
          pltpu.VMEM((c,), jnp.int32),
          pltpu.VMEM((c,), jnp.int32),
          pltpu.VMEM((c, h), jnp.float32),
          pltpu.VMEM((c, h), jnp.float32),
          pltpu.VMEM((c, h), jnp.float32),
          pltpu.VMEM((c, h), jnp.float32),
          pltpu.VMEM((c, h), jnp.float32),
          pltpu.VMEM((c, h), jnp.float32),
          pltpu.SemaphoreType.DMA,
          pltpu.SemaphoreType.DMA,
          pltpu.SemaphoreType.DMA,
          pltpu.SemaphoreType.DMA,
          pltpu.SemaphoreType.DMA,
          pltpu.SemaphoreType.DMA,
          pltpu.SemaphoreType.DMA,
          pltpu.SemaphoreType.DMA,
          pltpu.SemaphoreType.DMA,
          pltpu.SemaphoreType.DMA,
      ],
  )

  csc = 80

  def make_scatter(sz):
    return pl.kernel(
        functools.partial(_sc_scatter_body, epw=sz // _NW, c=csc, n=n, cz=400),
        out_type=jax.ShapeDtypeStruct((_NC, n, h), jnp.float32),
        mesh=mesh,
        scratch_types=[
            pltpu.VMEM((csc,), jnp.int32),
            pltpu.VMEM((csc,), jnp.int32),
            pltpu.VMEM((csc, h), jnp.float32),
            pltpu.VMEM((csc, h), jnp.float32),
            pltpu.VMEM_SHARED((n, h), jnp.float32),
            pltpu.SemaphoreType.DMA,
            pltpu.SemaphoreType.DMA,
            pltpu.SemaphoreType.DMA,
            pltpu.SemaphoreType.DMA,
        ],
    )

  for hd in halves:
    hd["gather"] = make_gather(hd["sz"])
    hd["scatter"] = make_scatter(hd["sz"])

  def make_msg(sz):
    return pl.pallas_call(
        _msg_body,
        grid=(sz // be,),
        in_specs=[
            pl.BlockSpec((be, h), lambda i: (i, 0)),
            pl.BlockSpec((1, 1, be), lambda i: (i, 0, 0)),
            full((nr, h)),
            full((h, h)),
            full((1, h)),
        ],
        out_specs=pl.BlockSpec((be, h), lambda i: (i, 0)),
        out_shape=jax.ShapeDtypeStruct((sz, h), jnp.float32),
    )

  for hd in halves:
    hd["msg"] = make_msg(hd["sz"])

  upd_call = pl.pallas_call(
      _upd_body,
      grid=(nbn,),
      in_specs=[
          pl.BlockSpec((2, bn, h), lambda i: (0, i, 0)),
          pl.BlockSpec((2, bn, h), lambda i: (0, i, 0)),
          pl.BlockSpec((bn, h), lambda i: (i, 0)),
          full((h, h)),
          full((1, h)),
          full((1, h)),
          full((1, h)),
          full((2, h, h)),
      ],
      out_specs=[
          pl.BlockSpec((bn, h), lambda i: (i, 0)),
          pl.BlockSpec((2, bn, h), lambda i: (0, i, 0)),
      ],
      out_shape=[
          jax.ShapeDtypeStruct((n, h), jnp.float32),
          jax.ShapeDtypeStruct((2, n, h), jnp.float32),
      ],
  )

  updlast_call = pl.pallas_call(
      _updlast_body,
      grid=(nbn,),
      in_specs=[
          pl.BlockSpec((2, bn, h), lambda i: (0, i, 0)),
          pl.BlockSpec((2, bn, h), lambda i: (0, i, 0)),
          pl.BlockSpec((bn, h), lambda i: (i, 0)),
          full((h, h)),
          full((1, h)),
          full((1, h)),
          full((1, h)),
      ],
      out_specs=[
          pl.BlockSpec((bn, h), lambda i: (i, 0)),
          full((1, h)),
      ],
      out_shape=[
          jax.ShapeDtypeStruct((n, h), jnp.float32),
          jax.ShapeDtypeStruct((1, h), jnp.float32),
      ],
  )

  hcur = h0
  for l in range(nlayers):
    t2 = tabs.reshape(2 * n, h)
    aggps = []
    for hd in halves:
      g = hd["gather"](t2, hd["src"], hd["dstn"])
      msg = hd["msg"](g, hd["rel3"], cp[l], edge_W2[l],
                      edge_b2[l].reshape(1, h))
      aggps.append(hd["scatter"](msg, hd["dst"], zeros_n))
    args = (aggps[0], aggps[1], hcur, node_W[l], node_b[l].reshape(1, h),
            ln_g[l].reshape(1, h), ln_b[l].reshape(1, h))
    if l + 1 < nlayers:
      hcur, tabs = upd_call(*args, edge_W1[l + 1, :2 * h, :].reshape(2, h, h))
    else:
      hcur, msum = updlast_call(*args)

  return (msum / n).reshape(h)
